# Initial kernel scaffold; baseline (speedup 1.0000x reference)
import jax
import jax.numpy as jnp
from jax.experimental import pallas as pl


def kernel(coords, bbox_min, bbox_max, tables):
    raise NotImplementedError("write your pallas kernel here")


# speedup vs baseline: 31.9624x; 31.9624x over previous
import numpy as np
import jax
import jax.numpy as jnp
from jax import lax
from jax.experimental import pallas as pl
from jax.experimental.pallas import tpu as pltpu
from jax.experimental.pallas import tpu_sc as plsc

_N_LEVELS = 16
_F = 2
_LOG2 = 19
_LOW = np.float32(16.0)
_HIGH = np.float32(512.0)
_N = 262144
_B = np.exp(np.log(_HIGH / _LOW) / np.float32(_N_LEVELS - 1))
_P1 = np.uint32(2654435761)
_P2 = np.uint32(805459861)
_MASK = np.uint32((1 << _LOG2) - 1)
_TABLE_ROWS = _N_LEVELS << _LOG2
_UNITS = _TABLE_ROWS // 4

_NC = 2
_NS = 16
_NW = _NC * _NS
_PPW = _N // _NW
_CH = 512
_NCHUNK = _PPW // _CH
_ROWS = 8 * _CH
_DMA_B = 128
_NDMA = _ROWS // _DMA_B


def _body(cx_h, cy_h, cz_h, const_h, tab_h, out_h, mask_h,
          constv, cxv, cyv, czv, wxv, wyv, wzv, maskv, idxu, idxf, embv, outv,
          sem_g):
    wid = lax.axis_index("s") * _NC + lax.axis_index("c")
    pltpu.sync_copy(const_h, constv)
    lanes = lax.broadcasted_iota(jnp.int32, (16,), 0)
    dup = lanes >> 1
    fpar = lanes & 1
    zero16 = jnp.zeros((16,), jnp.int32)
    one16 = jnp.full((16,), 1, jnp.int32)

    @pl.loop(0, _NCHUNK)
    def _chunk(ci):
        gbase = wid * _PPW + ci * _CH
        pltpu.sync_copy(cx_h.at[pl.ds(gbase, _CH)], cxv)
        pltpu.sync_copy(cy_h.at[pl.ds(gbase, _CH)], cyv)
        pltpu.sync_copy(cz_h.at[pl.ds(gbase, _CH)], czv)

        bminx = constv[pl.ds(3 * 16, 16)]
        bminy = constv[pl.ds(4 * 16, 16)]
        bminz = constv[pl.ds(5 * 16, 16)]
        bmaxx = constv[pl.ds(6 * 16, 16)]
        bmaxy = constv[pl.ds(7 * 16, 16)]
        bmaxz = constv[pl.ds(8 * 16, 16)]

        @pl.loop(0, _CH // 16)
        def _mask(i):
            p0 = i * 16
            x = cxv[pl.ds(p0, 16)]
            y = cyv[pl.ds(p0, 16)]
            z = czv[pl.ds(p0, 16)]
            kx = x == jnp.maximum(jnp.minimum(x, bmaxx), bminx)
            ky = y == jnp.maximum(jnp.minimum(y, bmaxy), bminy)
            kz = z == jnp.maximum(jnp.minimum(z, bmaxz), bminz)
            maskv[pl.ds(p0, 16)] = jnp.where(
                kx, jnp.where(ky, jnp.where(kz, one16, zero16), zero16), zero16)

        @pl.loop(0, _N_LEVELS)
        def _level(l):
            lful = jnp.full((16,), l, jnp.int32)
            gsx = plsc.load_gather(constv, [lful])
            gsy = plsc.load_gather(constv, [lful + 16])
            gsz = plsc.load_gather(constv, [lful + 32])
            off = l << _LOG2

            @pl.loop(0, _CH // 16)
            def _p1(i):
                p0 = i * 16
                x = cxv[pl.ds(p0, 16)]
                y = cyv[pl.ds(p0, 16)]
                z = czv[pl.ds(p0, 16)]
                xc = jnp.minimum(jnp.maximum(x, bminx), bmaxx)
                yc = jnp.minimum(jnp.maximum(y, bminy), bmaxy)
                zc = jnp.minimum(jnp.maximum(z, bminz), bmaxz)
                blx = ((xc - bminx) / gsx).astype(jnp.int32)
                bly = ((yc - bminy) / gsy).astype(jnp.int32)
                blz = ((zc - bminz) / gsz).astype(jnp.int32)
                vminx = blx.astype(jnp.float32) * gsx + bminx
                vminy = bly.astype(jnp.float32) * gsy + bminy
                vminz = blz.astype(jnp.float32) * gsz + bminz
                wxv[pl.ds(p0, 16)] = (x - vminx) / ((vminx + gsx) - vminx)
                wyv[pl.ds(p0, 16)] = (y - vminy) / ((vminy + gsy) - vminy)
                wzv[pl.ds(p0, 16)] = (z - vminz) / ((vminz + gsz) - vminz)
                xu0 = blx.astype(jnp.uint32)
                xu1 = xu0 + np.uint32(1)
                yu0 = bly.astype(jnp.uint32) * _P1
                yu1 = yu0 + _P1
                zu0 = blz.astype(jnp.uint32) * _P2
                zu1 = zu0 + _P2
                c = 0
                for xu in (xu0, xu1):
                    for yu in (yu0, yu1):
                        for zu in (zu0, zu1):
                            row = ((xu ^ yu ^ zu) & _MASK).astype(jnp.int32) + off
                            q = c * _CH + p0
                            idxu[q // _DMA_B, pl.ds(q % _DMA_B, 16)] = row >> 2
                            idxf[pl.ds(q, 16)] = row
                            c += 1

            @pl.loop(0, _NDMA)
            def _fire(j):
                pltpu.async_copy(
                    tab_h.at[idxu.at[j]],
                    embv.at[pl.ds(j * _DMA_B, _DMA_B), :],
                    sem_g)

            @pl.loop(0, _NDMA)
            def _drain(j):
                pltpu.make_async_copy(
                    tab_h.at[idxu.at[j]],
                    embv.at[pl.ds(j * _DMA_B, _DMA_B), :],
                    sem_g).wait()

            @pl.loop(0, _CH // 8)
            def _p3(k):
                p0 = k * 8
                rb = p0 + dup
                col = 2 * l + fpar
                wx = plsc.load_gather(wxv, [rb])
                wy = plsc.load_gather(wyv, [rb])
                wz = plsc.load_gather(wzv, [rb])
                e = []
                for c in range(8):
                    rc = rb + c * _CH
                    full = plsc.load_gather(idxf, [rc])
                    sub = ((full & 3) << 1) + fpar
                    e.append(plsc.load_gather(embv, [rc, sub]))
                omx = 1.0 - wx
                omy = 1.0 - wy
                omz = 1.0 - wz
                c00 = e[0] * omx + e[4] * wx
                c01 = e[1] * omx + e[5] * wx
                c10 = e[2] * omx + e[6] * wx
                c11 = e[3] * omx + e[7] * wx
                c0 = c00 * omy + c10 * wy
                c1 = c01 * omy + c11 * wy
                o = c0 * omz + c1 * wz
                plsc.store_scatter(outv, [rb, col], o)

        pltpu.sync_copy(outv, out_h.at[pl.ds(gbase, _CH)])
        pltpu.sync_copy(maskv, mask_h.at[pl.ds(gbase, _CH)])


def kernel(coords, bbox_min, bbox_max, tables):
    res = jnp.stack([
        jnp.floor(jnp.float32(_LOW) * jnp.float32(_B) ** i)
        for i in range(_N_LEVELS)
    ])
    gs = (bbox_max - bbox_min)[None, :] / res[:, None]
    const = jnp.zeros((12, 16), jnp.float32)
    const = const.at[0].set(gs[:, 0]).at[1].set(gs[:, 1]).at[2].set(gs[:, 2])
    const = const.at[3].set(bbox_min[0]).at[4].set(bbox_min[1]).at[5].set(bbox_min[2])
    const = const.at[6].set(bbox_max[0]).at[7].set(bbox_max[1]).at[8].set(bbox_max[2])
    const = const.reshape(-1)

    cx = coords[:, 0]
    cy = coords[:, 1]
    cz = coords[:, 2]
    tab8 = tables.reshape(_UNITS, 4 * _F)

    mesh = plsc.VectorSubcoreMesh(core_axis_name="c", subcore_axis_name="s")
    out, mask_i32 = pl.kernel(
        _body,
        out_type=(
            jax.ShapeDtypeStruct((_N, 2 * _N_LEVELS), jnp.float32),
            jax.ShapeDtypeStruct((_N,), jnp.int32),
        ),
        mesh=mesh,
        compiler_params=pltpu.CompilerParams(
            use_tc_tiling_on_sc=False, needs_layout_passes=False),
        scratch_types=[
            pltpu.VMEM((12 * 16,), jnp.float32),
            pltpu.VMEM((_CH,), jnp.float32),
            pltpu.VMEM((_CH,), jnp.float32),
            pltpu.VMEM((_CH,), jnp.float32),
            pltpu.VMEM((_CH,), jnp.float32),
            pltpu.VMEM((_CH,), jnp.float32),
            pltpu.VMEM((_CH,), jnp.float32),
            pltpu.VMEM((_CH,), jnp.int32),
            pltpu.VMEM((_NDMA, _DMA_B), jnp.int32),
            pltpu.VMEM((_ROWS,), jnp.int32),
            pltpu.VMEM((_ROWS, 4 * _F), jnp.float32),
            pltpu.VMEM((_CH, 2 * _N_LEVELS), jnp.float32),
            pltpu.SemaphoreType.DMA,
        ],
    )(cx, cy, cz, const, tab8)
    return out, mask_i32 != 0


# speedup vs baseline: 115.5314x; 3.6146x over previous
import numpy as np
import jax
import jax.numpy as jnp
from jax import lax
from jax.experimental import pallas as pl
from jax.experimental.pallas import tpu as pltpu
from jax.experimental.pallas import tpu_sc as plsc

_N_LEVELS = 16
_F = 2
_LOG2 = 19
_LOW = np.float32(16.0)
_HIGH = np.float32(512.0)
_N = 262144
_B = np.exp(np.log(_HIGH / _LOW) / np.float32(_N_LEVELS - 1))
_P1 = np.uint32(2654435761)
_P2 = np.uint32(805459861)
_MASK = np.uint32((1 << _LOG2) - 1)
_UNITS = (_N_LEVELS << (_LOG2 + 1)) // 8

_NC = 2
_NS = 16
_NW = _NC * _NS
_PPW = _N // _NW
_CH = 512
_NCHUNK = _PPW // _CH
_NIDX = 8 * _CH
_DMA_B = 128
_NDMA = 2 * _NIDX // _DMA_B


def _body(cx_h, cy_h, cz_h, const_h, tab_h, out_h, mask_h,
          constv, cxv, cyv, czv, wxv, wyv, wzv, maskv, idxu, idxs, embv, outv,
          sem_g):
    wid = lax.axis_index("s") * _NC + lax.axis_index("c")
    pltpu.sync_copy(const_h, constv)
    lanes = lax.broadcasted_iota(jnp.int32, (16,), 0)
    dup = lanes >> 1
    fpar = lanes & 1
    fp_row = fpar * _NIDX
    zero16 = jnp.zeros((16,), jnp.int32)
    one16 = jnp.full((16,), 1, jnp.int32)

    @pl.loop(0, _NCHUNK)
    def _chunk(ci):
        gbase = wid * _PPW + ci * _CH
        pltpu.sync_copy(cx_h.at[pl.ds(gbase, _CH)], cxv)
        pltpu.sync_copy(cy_h.at[pl.ds(gbase, _CH)], cyv)
        pltpu.sync_copy(cz_h.at[pl.ds(gbase, _CH)], czv)

        bminx = constv[pl.ds(3 * 16, 16)]
        bminy = constv[pl.ds(4 * 16, 16)]
        bminz = constv[pl.ds(5 * 16, 16)]
        bmaxx = constv[pl.ds(6 * 16, 16)]
        bmaxy = constv[pl.ds(7 * 16, 16)]
        bmaxz = constv[pl.ds(8 * 16, 16)]

        @pl.loop(0, _CH // 16)
        def _mask(i):
            p0 = i * 16
            x = cxv[pl.ds(p0, 16)]
            y = cyv[pl.ds(p0, 16)]
            z = czv[pl.ds(p0, 16)]
            kx = x == jnp.maximum(jnp.minimum(x, bmaxx), bminx)
            ky = y == jnp.maximum(jnp.minimum(y, bmaxy), bminy)
            kz = z == jnp.maximum(jnp.minimum(z, bmaxz), bminz)
            maskv[pl.ds(p0, 16)] = jnp.where(
                kx, jnp.where(ky, jnp.where(kz, one16, zero16), zero16), zero16)

        @pl.loop(0, _N_LEVELS)
        def _level(l):
            lful = jnp.full((16,), l, jnp.int32)
            gsx = plsc.load_gather(constv, [lful])
            gsy = plsc.load_gather(constv, [lful + 16])
            gsz = plsc.load_gather(constv, [lful + 32])
            ubase = l << 17

            @pl.loop(0, _CH // 16)
            def _p1(i):
                p0 = i * 16
                x = cxv[pl.ds(p0, 16)]
                y = cyv[pl.ds(p0, 16)]
                z = czv[pl.ds(p0, 16)]
                xc = jnp.minimum(jnp.maximum(x, bminx), bmaxx)
                yc = jnp.minimum(jnp.maximum(y, bminy), bmaxy)
                zc = jnp.minimum(jnp.maximum(z, bminz), bmaxz)
                blx = ((xc - bminx) / gsx).astype(jnp.int32)
                bly = ((yc - bminy) / gsy).astype(jnp.int32)
                blz = ((zc - bminz) / gsz).astype(jnp.int32)
                vminx = blx.astype(jnp.float32) * gsx + bminx
                vminy = bly.astype(jnp.float32) * gsy + bminy
                vminz = blz.astype(jnp.float32) * gsz + bminz
                wxv[pl.ds(p0, 16)] = (x - vminx) / ((vminx + gsx) - vminx)
                wyv[pl.ds(p0, 16)] = (y - vminy) / ((vminy + gsy) - vminy)
                wzv[pl.ds(p0, 16)] = (z - vminz) / ((vminz + gsz) - vminz)
                xu0 = blx.astype(jnp.uint32)
                xu1 = xu0 + np.uint32(1)
                yu0 = bly.astype(jnp.uint32) * _P1
                yu1 = yu0 + _P1
                zu0 = blz.astype(jnp.uint32) * _P2
                zu1 = zu0 + _P2
                c = 0
                for xu in (xu0, xu1):
                    for yu in (yu0, yu1):
                        for zu in (zu0, zu1):
                            h = ((xu ^ yu ^ zu) & _MASK).astype(jnp.int32)
                            u0 = ubase + ((h >> 2) & -32) + ((h >> 3) & 15)
                            q = c * _CH + p0
                            idxu[q // _DMA_B, pl.ds(q % _DMA_B, 16)] = u0
                            idxu[(_NIDX + q) // _DMA_B,
                                 pl.ds(q % _DMA_B, 16)] = u0 + 16
                            idxs[pl.ds(q, 16)] = h & 7
                            c += 1

            @pl.loop(0, _NDMA)
            def _fire(j):
                pltpu.async_copy(
                    tab_h.at[idxu.at[j]],
                    embv.at[pl.ds(j * _DMA_B, _DMA_B), :],
                    sem_g)

            @pl.loop(0, _NDMA)
            def _drain(j):
                pltpu.make_async_copy(
                    tab_h.at[idxu.at[j]],
                    embv.at[pl.ds(j * _DMA_B, _DMA_B), :],
                    sem_g).wait()

            @pl.loop(0, _CH // 8)
            def _p3(k):
                p0 = k * 8
                rb = p0 + dup
                col = 2 * l + fpar
                wx = plsc.load_gather(wxv, [rb])
                wy = plsc.load_gather(wyv, [rb])
                wz = plsc.load_gather(wzv, [rb])
                e = []
                for c in range(8):
                    rc = rb + c * _CH
                    sub = plsc.load_gather(idxs, [rc])
                    e.append(plsc.load_gather(embv, [rc + fp_row, sub]))
                omx = 1.0 - wx
                omy = 1.0 - wy
                omz = 1.0 - wz
                c00 = e[0] * omx + e[4] * wx
                c01 = e[1] * omx + e[5] * wx
                c10 = e[2] * omx + e[6] * wx
                c11 = e[3] * omx + e[7] * wx
                c0 = c00 * omy + c10 * wy
                c1 = c01 * omy + c11 * wy
                o = c0 * omz + c1 * wz
                plsc.store_scatter(outv, [rb, col], o)

        pltpu.sync_copy(outv, out_h.at[pl.ds(gbase, _CH)])
        pltpu.sync_copy(maskv, mask_h.at[pl.ds(gbase, _CH)])


def kernel(coords, bbox_min, bbox_max, tables):
    res = jnp.stack([
        jnp.floor(jnp.float32(_LOW) * jnp.float32(_B) ** i)
        for i in range(_N_LEVELS)
    ])
    gs = (bbox_max - bbox_min)[None, :] / res[:, None]
    const = jnp.zeros((12, 16), jnp.float32)
    const = const.at[0].set(gs[:, 0]).at[1].set(gs[:, 1]).at[2].set(gs[:, 2])
    const = const.at[3].set(bbox_min[0]).at[4].set(bbox_min[1]).at[5].set(bbox_min[2])
    const = const.at[6].set(bbox_max[0]).at[7].set(bbox_max[1]).at[8].set(bbox_max[2])
    const = const.reshape(-1)

    cx = coords[:, 0]
    cy = coords[:, 1]
    cz = coords[:, 2]
    tabv = tables.reshape(_N_LEVELS, 4096, 128, _F)
    tabv = tabv.transpose(0, 1, 3, 2).reshape(_UNITS, 8)

    mesh = plsc.VectorSubcoreMesh(core_axis_name="c", subcore_axis_name="s")
    out, mask_i32 = pl.kernel(
        _body,
        out_type=(
            jax.ShapeDtypeStruct((_N, 2 * _N_LEVELS), jnp.float32),
            jax.ShapeDtypeStruct((_N,), jnp.int32),
        ),
        mesh=mesh,
        compiler_params=pltpu.CompilerParams(
            use_tc_tiling_on_sc=False, needs_layout_passes=False),
        scratch_types=[
            pltpu.VMEM((12 * 16,), jnp.float32),
            pltpu.VMEM((_CH,), jnp.float32),
            pltpu.VMEM((_CH,), jnp.float32),
            pltpu.VMEM((_CH,), jnp.float32),
            pltpu.VMEM((_CH,), jnp.float32),
            pltpu.VMEM((_CH,), jnp.float32),
            pltpu.VMEM((_CH,), jnp.float32),
            pltpu.VMEM((_CH,), jnp.int32),
            pltpu.VMEM((_NDMA, _DMA_B), jnp.int32),
            pltpu.VMEM((_NIDX,), jnp.int32),
            pltpu.VMEM((2 * _NIDX, 8), jnp.float32),
            pltpu.VMEM((_CH, 2 * _N_LEVELS), jnp.float32),
            pltpu.SemaphoreType.DMA,
        ],
    )(cx, cy, cz, const, tabv)
    return out, mask_i32 != 0
